# speedup vs baseline: 6.0492x; 6.0492x over previous
import jax
import jax.numpy as jnp
from jax import lax
from jax.experimental import pallas as pl
from jax.experimental.pallas import tpu as pltpu
from jax.experimental.pallas import tpu_sc as plsc

B = 4
L = 8192
D = 128
NC = 2
NS = 16
NW = NC * NS
LPW = L // NW
CHUNK = 128
CPB = LPW // CHUNK
NCH = B * CPB
LANES = D // 16


def _body(ids_hbm, embed_hbm, type_hbm, pos_hbm, out_hbm,
          idx_v, type_v, bias_v, gb0, gb1, sg0, sg1, ss0, ss1):
    wid = lax.axis_index("s") * NC + lax.axis_index("c")
    l_base = wid * LPW

    for b in range(B):
        pltpu.sync_copy(ids_hbm.at[pl.ds(b * L + l_base, LPW)],
                        idx_v.at[pl.ds(b * LPW, LPW)])

    pltpu.sync_copy(type_hbm.at[0], type_v)
    pltpu.sync_copy(pos_hbm.at[pl.ds(l_base, LPW)], bias_v)
    tvecs = [type_v[pl.ds(j * 16, 16)] for j in range(LANES)]

    def add_type(r, carry):
        for j in range(LANES):
            sl = pl.ds(j * 16, 16)
            bias_v[r, sl] = bias_v[r, sl] + tvecs[j]
        return carry

    lax.fori_loop(0, LPW, add_type, 0)

    gbufs = (gb0, gb1)
    gsems = (sg0, sg1)
    ssems = (ss0, ss1)

    def issue_gather(k, slot):
        idx_ref = idx_v.at[pl.ds(k * CHUNK, CHUNK)]
        return pltpu.async_copy(embed_hbm.at[idx_ref], gbufs[slot], gsems[slot])

    g_handles = [None, None]
    s_handles = [None, None]
    g_handles[0] = issue_gather(0, 0)
    for k in range(NCH):
        cb = k % 2
        nb = (k + 1) % 2
        if k + 1 < NCH:
            if s_handles[nb] is not None:
                s_handles[nb].wait()
                s_handles[nb] = None
            g_handles[nb] = issue_gather(k + 1, nb)
        g_handles[cb].wait()

        buf = gbufs[cb]
        boff = (k % CPB) * CHUNK

        def row_body(r, carry):
            for j in range(LANES):
                sl = pl.ds(j * 16, 16)
                plsc.addupdate(buf.at[r, sl], bias_v[boff + r, sl])
            return carry

        lax.fori_loop(0, CHUNK, row_body, 0)

        row0 = (k // CPB) * L + l_base + (k % CPB) * CHUNK
        s_handles[cb] = pltpu.async_copy(
            buf, out_hbm.at[pl.ds(row0, CHUNK)], ssems[cb])

    s_handles[0].wait()
    s_handles[1].wait()


_emb_lookup = pl.kernel(
    _body,
    out_type=jax.ShapeDtypeStruct((B * L, D), jnp.float32),
    mesh=plsc.VectorSubcoreMesh(core_axis_name="c", subcore_axis_name="s",
                                num_cores=NC, num_subcores=NS),
    scratch_types=[
        pltpu.VMEM((B * LPW,), jnp.int32),
        pltpu.VMEM((D,), jnp.float32),
        pltpu.VMEM((LPW, D), jnp.float32),
        pltpu.VMEM((CHUNK, D), jnp.float32),
        pltpu.VMEM((CHUNK, D), jnp.float32),
        pltpu.SemaphoreType.DMA,
        pltpu.SemaphoreType.DMA,
        pltpu.SemaphoreType.DMA,
        pltpu.SemaphoreType.DMA,
    ],
)


@jax.jit
def kernel(input_ids, embed_weight, type_weight, pos_weight):
    ids_flat = input_ids.reshape(-1).astype(jnp.int32)
    out = _emb_lookup(ids_flat, embed_weight, type_weight, pos_weight)
    return out.reshape(B, L, D)


# speedup vs baseline: 6.8742x; 1.1364x over previous
import jax
import jax.numpy as jnp
from jax import lax
from jax.experimental import pallas as pl
from jax.experimental.pallas import tpu as pltpu
from jax.experimental.pallas import tpu_sc as plsc

B = 4
L = 8192
D = 128
NC = 2
NS = 16
NW = NC * NS
LPW = L // NW
CHUNK = 32
NG = LPW // CHUNK
NBG = 3
NBO = 2
LANES = D // 16


def _body(ids_hbm, embed_hbm, type_hbm, pos_hbm, out_hbm,
          idx_v, type_v, pos_v, gbuf, obuf,
          sem_idx, sem_pt, sg0, sg1, sg2, ss0, ss1):
    wid = lax.axis_index("s") * NC + lax.axis_index("c")
    l_base = wid * LPW

    idx_h = [pltpu.async_copy(ids_hbm.at[b, pl.ds(l_base, LPW)],
                              idx_v.at[b], sem_idx)
             for b in range(B)]
    pos_h = pltpu.async_copy(pos_hbm.at[pl.ds(l_base, LPW)], pos_v, sem_pt)
    typ_h = pltpu.async_copy(type_hbm.at[0], type_v, sem_pt)
    for h in idx_h:
        h.wait()

    sg = (sg0, sg1, sg2)
    ss = (ss0, ss1)

    def issue_gathers(g):
        par = g % NBG
        return [pltpu.async_copy(
                    embed_hbm.at[idx_v.at[b, pl.ds(g * CHUNK, CHUNK)]],
                    gbuf.at[par, b], sg[par])
                for b in range(B)]

    g_handles = [None] * NBG
    s_handles = [None] * NBO
    for g in range(NBG):
        g_handles[g] = issue_gathers(g)

    pos_h.wait()
    typ_h.wait()
    tvecs = [type_v[pl.ds(j * 16, 16)] for j in range(LANES)]

    for g in range(NG):
        gpar = g % NBG
        opar = g % NBO
        if s_handles[opar] is not None:
            for h in s_handles[opar]:
                h.wait()
            s_handles[opar] = None
        for h in g_handles[gpar]:
            h.wait()

        boff = g * CHUNK

        def row_body(r, carry):
            bias = [pos_v[boff + r, pl.ds(j * 16, 16)] + tvecs[j]
                    for j in range(LANES)]
            for b in range(B):
                for j in range(LANES):
                    sl = pl.ds(j * 16, 16)
                    obuf[opar, b, r, sl] = gbuf[gpar, b, r, sl] + bias[j]
            return carry

        lax.fori_loop(0, CHUNK, row_body, 0)

        if g + NBG < NG:
            g_handles[gpar] = issue_gathers(g + NBG)

        s_handles[opar] = [
            pltpu.async_copy(
                obuf.at[opar, b],
                out_hbm.at[b, pl.ds(l_base + g * CHUNK, CHUNK)],
                ss[opar])
            for b in range(B)]

    for hs in s_handles:
        for h in hs:
            h.wait()


_emb_lookup = pl.kernel(
    _body,
    out_type=jax.ShapeDtypeStruct((B, L, D), jnp.float32),
    mesh=plsc.VectorSubcoreMesh(core_axis_name="c", subcore_axis_name="s",
                                num_cores=NC, num_subcores=NS),
    scratch_types=[
        pltpu.VMEM((B, LPW), jnp.int32),
        pltpu.VMEM((D,), jnp.float32),
        pltpu.VMEM((LPW, D), jnp.float32),
        pltpu.VMEM((NBG, B, CHUNK, D), jnp.float32),
        pltpu.VMEM((NBO, B, CHUNK, D), jnp.float32),
        pltpu.SemaphoreType.DMA,
        pltpu.SemaphoreType.DMA,
        pltpu.SemaphoreType.DMA,
        pltpu.SemaphoreType.DMA,
        pltpu.SemaphoreType.DMA,
        pltpu.SemaphoreType.DMA,
        pltpu.SemaphoreType.DMA,
    ],
)


@jax.jit
def kernel(input_ids, embed_weight, type_weight, pos_weight):
    return _emb_lookup(input_ids.astype(jnp.int32),
                       embed_weight, type_weight, pos_weight)


# speedup vs baseline: 6.9974x; 1.0179x over previous
import jax
import jax.numpy as jnp
from jax import lax
from jax.experimental import pallas as pl
from jax.experimental.pallas import tpu as pltpu
from jax.experimental.pallas import tpu_sc as plsc

B = 4
L = 8192
D = 128
NC = 2
NS = 16
NW = NC * NS
LPW = L // NW
CHUNK = 32
GROWS = B * CHUNK
NG = LPW // CHUNK
NBG = 3
NBO = 2
LANES = D // 16


def _body(ids_hbm, embed_hbm, type_hbm, pos_hbm, out_hbm,
          idx_raw, gidx, type_v, pos_v, gbuf, obuf,
          sem_idx, sem_pt, sg0, sg1, sg2, ss0, ss1):
    wid = lax.axis_index("s") * NC + lax.axis_index("c")
    l_base = wid * LPW

    idx_h = [pltpu.async_copy(ids_hbm.at[b, pl.ds(l_base, LPW)],
                              idx_raw.at[b], sem_idx)
             for b in range(B)]
    pos_h = pltpu.async_copy(pos_hbm.at[pl.ds(l_base, LPW)], pos_v, sem_pt)
    typ_h = pltpu.async_copy(type_hbm.at[0], type_v, sem_pt)
    for h in idx_h:
        h.wait()

    for g in range(NG):
        for b in range(B):
            for j in range(CHUNK // 16):
                gidx[g, pl.ds(b * CHUNK + j * 16, 16)] = (
                    idx_raw[b, pl.ds(g * CHUNK + j * 16, 16)])

    sg = (sg0, sg1, sg2)
    ss = (ss0, ss1)

    def issue_gather(g):
        par = g % NBG
        return pltpu.async_copy(embed_hbm.at[gidx.at[g]], gbuf.at[par],
                                sg[par])

    g_handles = [None] * NBG
    s_handles = [None] * NBO
    for g in range(NBG):
        g_handles[g] = issue_gather(g)

    pos_h.wait()
    typ_h.wait()
    tvecs = [type_v[pl.ds(j * 16, 16)] for j in range(LANES)]

    for g in range(NG):
        gpar = g % NBG
        opar = g % NBO
        if s_handles[opar] is not None:
            s_handles[opar].wait()
            s_handles[opar] = None
        g_handles[gpar].wait()

        boff = g * CHUNK

        def row_body(r, carry):
            bias = [pos_v[boff + r, pl.ds(j * 16, 16)] + tvecs[j]
                    for j in range(LANES)]
            for b in range(B):
                for j in range(LANES):
                    sl = pl.ds(j * 16, 16)
                    obuf[opar, b, r, sl] = gbuf[gpar, b * CHUNK + r, sl] + bias[j]
            return carry

        lax.fori_loop(0, CHUNK, row_body, 0)

        if g + NBG < NG:
            g_handles[gpar] = issue_gather(g + NBG)

        s_handles[opar] = pltpu.async_copy(
            obuf.at[opar],
            out_hbm.at[pl.ds(0, B), pl.ds(l_base + g * CHUNK, CHUNK)],
            ss[opar])

    for h in s_handles:
        h.wait()


_emb_lookup = pl.kernel(
    _body,
    out_type=jax.ShapeDtypeStruct((B, L, D), jnp.float32),
    mesh=plsc.VectorSubcoreMesh(core_axis_name="c", subcore_axis_name="s",
                                num_cores=NC, num_subcores=NS),
    scratch_types=[
        pltpu.VMEM((B, LPW), jnp.int32),
        pltpu.VMEM((NG, GROWS), jnp.int32),
        pltpu.VMEM((D,), jnp.float32),
        pltpu.VMEM((LPW, D), jnp.float32),
        pltpu.VMEM((NBG, GROWS, D), jnp.float32),
        pltpu.VMEM((NBO, B, CHUNK, D), jnp.float32),
        pltpu.SemaphoreType.DMA,
        pltpu.SemaphoreType.DMA,
        pltpu.SemaphoreType.DMA,
        pltpu.SemaphoreType.DMA,
        pltpu.SemaphoreType.DMA,
        pltpu.SemaphoreType.DMA,
        pltpu.SemaphoreType.DMA,
    ],
)


@jax.jit
def kernel(input_ids, embed_weight, type_weight, pos_weight):
    return _emb_lookup(input_ids.astype(jnp.int32),
                       embed_weight, type_weight, pos_weight)


# speedup vs baseline: 7.5363x; 1.0770x over previous
import jax
import jax.numpy as jnp
from jax import lax
from jax.experimental import pallas as pl
from jax.experimental.pallas import tpu as pltpu
from jax.experimental.pallas import tpu_sc as plsc

B = 4
L = 8192
D = 128
NC = 2
NS = 16
NW = NC * NS
LPW = L // NW
CHUNK = 32
GROWS = B * CHUNK
NG = LPW // CHUNK
NBG = 5
LANES = D // 16


def _body(ids_hbm, embed_hbm, type_hbm, pos_hbm, out_hbm,
          idx_raw, gidx, type_v, pos_v, gbuf,
          sem_idx, sem_pt, sg0, sg1, sg2, sg3, sg4,
          ss0, ss1, ss2, ss3, ss4):
    wid = lax.axis_index("s") * NC + lax.axis_index("c")
    l_base = wid * LPW

    idx_h = [pltpu.async_copy(ids_hbm.at[b, pl.ds(l_base, LPW)],
                              idx_raw.at[b], sem_idx)
             for b in range(B)]
    pos_h = pltpu.async_copy(pos_hbm.at[pl.ds(l_base, LPW)], pos_v, sem_pt)
    typ_h = pltpu.async_copy(type_hbm.at[0], type_v, sem_pt)
    for h in idx_h:
        h.wait()

    for g in range(NG):
        for b in range(B):
            for j in range(CHUNK // 16):
                gidx[g, pl.ds(b * CHUNK + j * 16, 16)] = (
                    idx_raw[b, pl.ds(g * CHUNK + j * 16, 16)])

    sg = (sg0, sg1, sg2, sg3, sg4)
    ss = (ss0, ss1, ss2, ss3, ss4)

    def issue_gather(g):
        par = g % NBG
        return pltpu.async_copy(embed_hbm.at[gidx.at[g]], gbuf.at[par],
                                sg[par])

    g_handles = [None] * NBG
    s_handles = [None] * NBG
    for g in range(NBG):
        g_handles[g] = issue_gather(g)

    pos_h.wait()
    typ_h.wait()
    tvecs = [type_v[pl.ds(j * 16, 16)] for j in range(LANES)]

    for g in range(NG):
        par = g % NBG
        g_handles[par].wait()

        boff = g * CHUNK

        def row_body(r, carry):
            bias = [pos_v[boff + r, pl.ds(j * 16, 16)] + tvecs[j]
                    for j in range(LANES)]
            for b in range(B):
                for j in range(LANES):
                    sl = pl.ds(j * 16, 16)
                    gbuf[par, b * CHUNK + r, sl] = (
                        gbuf[par, b * CHUNK + r, sl] + bias[j])
            return carry

        lax.fori_loop(0, CHUNK, row_body, 0)

        if g + NBG < NG:
            if s_handles[par] is not None:
                s_handles[par].wait()
                s_handles[par] = None
            g_handles[par] = issue_gather(g + NBG)

        s_handles[par] = pltpu.async_copy(
            gbuf.at[par].reshape(B, CHUNK, D),
            out_hbm.at[pl.ds(0, B), pl.ds(l_base + g * CHUNK, CHUNK)],
            ss[par])

    for h in s_handles:
        if h is not None:
            h.wait()


_emb_lookup = pl.kernel(
    _body,
    out_type=jax.ShapeDtypeStruct((B, L, D), jnp.float32),
    mesh=plsc.VectorSubcoreMesh(core_axis_name="c", subcore_axis_name="s",
                                num_cores=NC, num_subcores=NS),
    scratch_types=[
        pltpu.VMEM((B, LPW), jnp.int32),
        pltpu.VMEM((NG, GROWS), jnp.int32),
        pltpu.VMEM((D,), jnp.float32),
        pltpu.VMEM((LPW, D), jnp.float32),
        pltpu.VMEM((NBG, GROWS, D), jnp.float32),
        pltpu.SemaphoreType.DMA,
        pltpu.SemaphoreType.DMA,
        pltpu.SemaphoreType.DMA,
        pltpu.SemaphoreType.DMA,
        pltpu.SemaphoreType.DMA,
        pltpu.SemaphoreType.DMA,
        pltpu.SemaphoreType.DMA,
        pltpu.SemaphoreType.DMA,
        pltpu.SemaphoreType.DMA,
        pltpu.SemaphoreType.DMA,
        pltpu.SemaphoreType.DMA,
        pltpu.SemaphoreType.DMA,
    ],
)


@jax.jit
def kernel(input_ids, embed_weight, type_weight, pos_weight):
    return _emb_lookup(input_ids.astype(jnp.int32),
                       embed_weight, type_weight, pos_weight)
